# baseline (speedup 1.0000x reference)
import functools

import jax
import jax.numpy as jnp
from jax import lax
from jax.experimental import pallas as pl
from jax.experimental.pallas import tpu as pltpu
from jax.experimental.pallas import tpu_sc as plsc

D = 16
NW = 32
CHUNK = 128
K = 13
OUTER = 8
N_CHUNKS = NW * OUTER * K


def _make_gather():
    mesh = plsc.VectorSubcoreMesh(core_axis_name="c", subcore_axis_name="s")
    out_type = jax.ShapeDtypeStruct((N_CHUNKS, CHUNK, D), jnp.float32)

    @functools.partial(
        pl.kernel,
        mesh=mesh,
        out_type=out_type,
        compiler_params=pltpu.CompilerParams(use_tc_tiling_on_sc=False),
        scratch_types=[
            pltpu.VMEM((OUTER, K, CHUNK), jnp.int32),
            pltpu.VMEM((K, CHUNK, D), jnp.float32),
            pltpu.SemaphoreType.DMA,
        ],
    )
    def gather_kernel(idx_hbm, table_hbm, out_hbm, idx_v, rows_v, sem):
        wid = lax.axis_index("s") * 2 + lax.axis_index("c")
        pltpu.sync_copy(idx_hbm.at[wid], idx_v)

        def body(c, _):
            cps = [
                pltpu.async_copy(table_hbm.at[idx_v.at[c, j]], rows_v.at[j], sem)
                for j in range(K)
            ]
            for cp in cps:
                cp.wait()
            pltpu.sync_copy(rows_v, out_hbm.at[pl.ds((wid * OUTER + c) * K, K)])
            return ()

        lax.fori_loop(0, OUTER, body, ())

    return gather_kernel


_gather = _make_gather()


def kernel(index, table):
    idx = index.reshape(NW, OUTER, K, CHUNK).astype(jnp.int32)
    out = _gather(idx, table)
    return out.reshape(16384, 26, D)


# speedup vs baseline: 1.0057x; 1.0057x over previous
import functools

import jax
import jax.numpy as jnp
from jax import lax
from jax.experimental import pallas as pl
from jax.experimental.pallas import tpu as pltpu
from jax.experimental.pallas import tpu_sc as plsc

D = 16
NW = 32
CHUNK = 128
K = 13
OUTER = 8
NBUF = 2
N_CHUNKS = NW * OUTER * K


def _make_gather():
    mesh = plsc.VectorSubcoreMesh(core_axis_name="c", subcore_axis_name="s")
    out_type = jax.ShapeDtypeStruct((N_CHUNKS, CHUNK, D), jnp.float32)

    @functools.partial(
        pl.kernel,
        mesh=mesh,
        out_type=out_type,
        compiler_params=pltpu.CompilerParams(use_tc_tiling_on_sc=False),
        scratch_types=[
            pltpu.VMEM((OUTER, K, CHUNK), jnp.int32),
            pltpu.VMEM((NBUF, K, CHUNK, D), jnp.float32),
            pltpu.SemaphoreType.DMA,
            pltpu.SemaphoreType.DMA,
        ],
    )
    def gather_kernel(idx_hbm, table_hbm, out_hbm, idx_v, rows_v, sem0, sem1):
        wid = lax.axis_index("s") * 2 + lax.axis_index("c")
        pltpu.sync_copy(idx_hbm.at[wid], idx_v)
        sems = (sem0, sem1)

        def fire(g, slot, sem):
            for j in range(K):
                pltpu.async_copy(
                    table_hbm.at[idx_v.at[g, j]], rows_v.at[slot, j], sem
                )

        def drain(slot, sem):
            pltpu.make_async_copy(
                out_hbm.at[pl.ds(0, K)], rows_v.at[slot], sem
            ).wait()

        fire(0, 0, sem0)

        def body(i, _):
            for b in range(NBUF):
                c = NBUF * i + b
                nb = 1 - b

                @pl.when(c + 1 < OUTER)
                def _():
                    fire(c + 1, nb, sems[nb])

                drain(b, sems[b])
                pltpu.sync_copy(
                    rows_v.at[b], out_hbm.at[pl.ds((wid * OUTER + c) * K, K)]
                )
            return ()

        lax.fori_loop(0, OUTER // NBUF, body, ())

    return gather_kernel


_gather = _make_gather()


def kernel(index, table):
    idx = index.reshape(NW, OUTER, K, CHUNK).astype(jnp.int32)
    out = _gather(idx, table)
    return out.reshape(16384, 26, D)


# speedup vs baseline: 1.1585x; 1.1519x over previous
import functools

import jax
import jax.numpy as jnp
from jax import lax
from jax.experimental import pallas as pl
from jax.experimental.pallas import tpu as pltpu
from jax.experimental.pallas import tpu_sc as plsc

D = 16
S = 26
NW = 32
ROWS_W = 512
RG = 128
G = ROWS_W // RG
NBUF = 2


def _make_gather():
    mesh = plsc.VectorSubcoreMesh(core_axis_name="c", subcore_axis_name="s")
    out_type = jax.ShapeDtypeStruct((16384, S, D), jnp.float32)

    @functools.partial(
        pl.kernel,
        mesh=mesh,
        out_type=out_type,
        compiler_params=pltpu.CompilerParams(use_tc_tiling_on_sc=False),
        scratch_types=[
            pltpu.VMEM((ROWS_W, S), jnp.int32),
            pltpu.VMEM((NBUF, RG, S, D), jnp.float32),
            pltpu.SemaphoreType.DMA,
            pltpu.SemaphoreType.DMA,
        ],
    )
    def gather_kernel(idx_hbm, table_hbm, out_hbm, idx_v, rows_v, sem0, sem1):
        wid = lax.axis_index("s") * 2 + lax.axis_index("c")
        base = wid * ROWS_W
        pltpu.sync_copy(idx_hbm.at[pl.ds(base, ROWS_W)], idx_v)
        sems = (sem0, sem1)

        def fire(g, slot, sem):
            def frow(r, _):
                pltpu.async_copy(
                    table_hbm.at[idx_v.at[g * RG + r]], rows_v.at[slot, r], sem
                )
                return ()

            lax.fori_loop(0, RG, frow, ())

        def drain(slot, sem):
            pltpu.make_async_copy(
                out_hbm.at[pl.ds(0, RG)], rows_v.at[slot], sem
            ).wait()

        fire(0, 0, sem0)

        def body(i, _):
            for b in range(NBUF):
                g = NBUF * i + b
                nb = 1 - b

                @pl.when(g + 1 < G)
                def _():
                    fire(g + 1, nb, sems[nb])

                drain(b, sems[b])
                pltpu.sync_copy(
                    rows_v.at[b], out_hbm.at[pl.ds(base + g * RG, RG)]
                )
            return ()

        lax.fori_loop(0, G // NBUF, body, ())

    return gather_kernel


_gather = _make_gather()


def kernel(index, table):
    return _gather(index.astype(jnp.int32), table)


# speedup vs baseline: 1.3611x; 1.1749x over previous
import functools

import jax
import jax.numpy as jnp
from jax import lax
from jax.experimental import pallas as pl
from jax.experimental.pallas import tpu as pltpu
from jax.experimental.pallas import tpu_sc as plsc

D = 16
S = 26
NW = 32
ROWS_W = 512
RG = 128
G = ROWS_W // RG
NBUF = 2


def _make_gather():
    mesh = plsc.VectorSubcoreMesh(core_axis_name="c", subcore_axis_name="s")
    out_type = jax.ShapeDtypeStruct((S, 2, NW * G, 8, RG), jnp.float32)

    @functools.partial(
        pl.kernel,
        mesh=mesh,
        out_type=out_type,
        compiler_params=pltpu.CompilerParams(
            use_tc_tiling_on_sc=False, needs_layout_passes=False
        ),
        scratch_types=[
            pltpu.VMEM((ROWS_W, S), jnp.int32),
            pltpu.VMEM((NBUF, RG * S, D), jnp.float32),
            pltpu.VMEM((D, RG), jnp.float32),
            pltpu.SemaphoreType.DMA,
            pltpu.SemaphoreType.DMA,
        ],
    )
    def gather_kernel(idx_hbm, table_hbm, out_hbm, idx_v, rows_v, blk_v, sem0, sem1):
        wid = lax.axis_index("s") * 2 + lax.axis_index("c")
        base = wid * ROWS_W
        pltpu.sync_copy(idx_hbm.at[pl.ds(base, ROWS_W)], idx_v)
        sems = (sem0, sem1)

        iota = lax.iota(jnp.int32, 16)

        def fire(g, slot, sem):
            def frow(r, _):
                pltpu.async_copy(
                    table_hbm.at[idx_v.at[g * RG + r]],
                    rows_v.at[slot, pl.ds(r * S, S)],
                    sem,
                )
                return ()

            lax.fori_loop(0, RG, frow, ())

        def drain(slot, sem):
            pltpu.make_async_copy(
                table_hbm.at[pl.ds(0, RG * S)], rows_v.at[slot], sem
            ).wait()

        def emit(g, slot):
            bt = wid * G + g

            def fs(s, _):
                def fb(bo, _):
                    for j in range(8):
                        b = bo * 8 + j
                        vec = rows_v[slot, b * S + s, :]
                        bvec = jnp.full((16,), b, jnp.int32)
                        plsc.store_scatter(blk_v, [iota, bvec], vec)
                    return ()

                lax.fori_loop(0, RG // 8, fb, ())
                pltpu.sync_copy(blk_v.at[pl.ds(0, 8)], out_hbm.at[s, 0, bt])
                pltpu.sync_copy(blk_v.at[pl.ds(8, 8)], out_hbm.at[s, 1, bt])
                return ()

            lax.fori_loop(0, S, fs, ())

        fire(0, 0, sem0)

        def body(i, _):
            for b in range(NBUF):
                g = NBUF * i + b
                nb = 1 - b

                @pl.when(g + 1 < G)
                def _():
                    fire(g + 1, nb, sems[nb])

                drain(b, sems[b])
                emit(g, b)
            return ()

        lax.fori_loop(0, G // NBUF, body, ())

    return gather_kernel


_gather = _make_gather()


def kernel(index, table):
    out5d = _gather(index.astype(jnp.int32), table)
    o = jnp.transpose(out5d, (2, 4, 0, 1, 3))
    return o.reshape(16384, S, D)


# speedup vs baseline: 2.4723x; 1.8164x over previous
import functools

import jax
import jax.numpy as jnp
from jax import lax
from jax.experimental import pallas as pl
from jax.experimental.pallas import tpu as pltpu
from jax.experimental.pallas import tpu_sc as plsc

D = 16
S = 26
NW = 32
ROWS_W = 512
RG = 128
G = ROWS_W // RG
NBUF = 2
NB_FULL = 7812
TAIL = 64


def _make_transpose():
    mesh = plsc.VectorSubcoreMesh(core_axis_name="c", subcore_axis_name="s")
    out_type = jax.ShapeDtypeStruct((16000000,), jnp.float32)

    @functools.partial(
        pl.kernel,
        mesh=mesh,
        out_type=out_type,
        compiler_params=pltpu.CompilerParams(
            use_tc_tiling_on_sc=True, needs_layout_passes=False
        ),
        scratch_types=[
            pltpu.VMEM((D, 128), jnp.float32),
            pltpu.VMEM((D, 128), jnp.float32),
            pltpu.VMEM((128 * D,), jnp.float32),
            pltpu.VMEM((128 * D,), jnp.float32),
            pltpu.SemaphoreType.DMA,
            pltpu.SemaphoreType.DMA,
        ],
    )
    def tr_kernel(tT_hbm, tail_hbm, out_hbm, buf0, buf1, tp0, tp1, sem_i, sem_o):
        bufs = (buf0, buf1)
        tps = (tp0, tp1)
        wid = lax.axis_index("s") * 2 + lax.axis_index("c")
        nblk = jnp.where(wid < 4, 245, 244)
        start = wid * 244 + jnp.minimum(wid, 4)
        iota16 = lax.iota(jnp.int32, 16) * D

        def fire_in(b, slot):
            pltpu.async_copy(
                tT_hbm.at[:, pl.ds(b * 128, 128)], bufs[slot], sem_i
            )

        def wait_in(slot):
            pltpu.make_async_copy(
                tT_hbm.at[:, pl.ds(0, 128)], bufs[slot], sem_i
            ).wait()

        def transpose(slot, njj):
            def fj(jj, _):
                for f in range(D):
                    vec = bufs[slot][f, pl.ds(jj * 16, 16)]
                    pos = iota16 + (jj * 256 + f)
                    plsc.store_scatter(tps[slot], [pos], vec)
                return ()

            lax.fori_loop(0, njj, fj, ())

        def fire_out(b, slot):
            pltpu.async_copy(
                tps[slot], out_hbm.at[pl.ds(b * 2048, 2048)], sem_o
            )

        def wait_out(slot):
            pltpu.make_async_copy(
                tps[slot], out_hbm.at[pl.ds(0, 2048)], sem_o
            ).wait()

        fire_in(start, 0)

        def body(i, _):
            for sl in range(NBUF):
                k = NBUF * i + sl
                b = start + k

                @pl.when(k < nblk)
                def _():
                    @pl.when(k + 1 < nblk)
                    def _():
                        fire_in(b + 1, 1 - sl)

                    wait_in(sl)

                    @pl.when(k >= 2)
                    def _():
                        wait_out(sl)

                    transpose(sl, 8)
                    fire_out(b, sl)

            return ()

        lax.fori_loop(0, 123, body, ())
        wait_out(0)
        wait_out(0)

        @pl.when(wid == 31)
        def _():
            pltpu.sync_copy(tail_hbm, tp0.at[pl.ds(0, TAIL * D)])
            pltpu.sync_copy(
                tp0.at[pl.ds(0, TAIL * D)],
                out_hbm.at[pl.ds(NB_FULL * 2048, TAIL * D)],
            )

    return tr_kernel


def _make_gather():
    mesh = plsc.VectorSubcoreMesh(core_axis_name="c", subcore_axis_name="s")
    out_type = jax.ShapeDtypeStruct((S, 2, NW * G, 8, RG), jnp.float32)

    @functools.partial(
        pl.kernel,
        mesh=mesh,
        out_type=out_type,
        compiler_params=pltpu.CompilerParams(
            use_tc_tiling_on_sc=False, needs_layout_passes=False
        ),
        scratch_types=[
            pltpu.VMEM((ROWS_W, S), jnp.int32),
            pltpu.VMEM((NBUF, RG * S, D), jnp.float32),
            pltpu.VMEM((D, RG), jnp.float32),
            pltpu.SemaphoreType.DMA,
            pltpu.SemaphoreType.DMA,
        ],
    )
    def gather_kernel(idx_hbm, table_hbm, out_hbm, idx_v, rows_v, blk_v, sem0, sem1):
        wid = lax.axis_index("s") * 2 + lax.axis_index("c")
        base = wid * ROWS_W
        pltpu.sync_copy(idx_hbm.at[pl.ds(base, ROWS_W)], idx_v)
        sems = (sem0, sem1)

        iota = lax.iota(jnp.int32, 16)

        def fire(g, slot, sem):
            def frow(r, _):
                pltpu.async_copy(
                    table_hbm.at[idx_v.at[g * RG + r]],
                    rows_v.at[slot, pl.ds(r * S, S)],
                    sem,
                )
                return ()

            lax.fori_loop(0, RG, frow, ())

        def drain(slot, sem):
            pltpu.make_async_copy(
                table_hbm.at[pl.ds(0, RG * S)], rows_v.at[slot], sem
            ).wait()

        def emit(g, slot):
            bt = wid * G + g

            def fs(s, _):
                def fb(bo, _):
                    for j in range(8):
                        b = bo * 8 + j
                        vec = rows_v[slot, b * S + s, :]
                        bvec = jnp.full((16,), b, jnp.int32)
                        plsc.store_scatter(blk_v, [iota, bvec], vec)
                    return ()

                lax.fori_loop(0, RG // 8, fb, ())
                pltpu.sync_copy(blk_v.at[pl.ds(0, 8)], out_hbm.at[s, 0, bt])
                pltpu.sync_copy(blk_v.at[pl.ds(8, 8)], out_hbm.at[s, 1, bt])
                return ()

            lax.fori_loop(0, S, fs, ())

        fire(0, 0, sem0)

        def body(i, _):
            for b in range(NBUF):
                g = NBUF * i + b
                nb = 1 - b

                @pl.when(g + 1 < G)
                def _():
                    fire(g + 1, nb, sems[nb])

                drain(b, sems[b])
                emit(g, b)
            return ()

        lax.fori_loop(0, G // NBUF, body, ())

    return gather_kernel


_transpose = _make_transpose()
_gather = _make_gather()


def kernel(index, table):
    tail = table[NB_FULL * 128 :].reshape(TAIL * D)
    t_lin = _transpose(table.T, tail)
    out5d = _gather(index.astype(jnp.int32), t_lin.reshape(1000000, 16))
    o = jnp.transpose(out5d, (2, 4, 0, 1, 3))
    return o.reshape(16384, S, D)


# speedup vs baseline: 2.6137x; 1.0572x over previous
import functools

import jax
import jax.numpy as jnp
from jax import lax
from jax.experimental import pallas as pl
from jax.experimental.pallas import tpu as pltpu
from jax.experimental.pallas import tpu_sc as plsc

D = 16
S = 26
NW = 32
ROWS_W = 512
RG = 128
G = ROWS_W // RG
NBUF = 2
NB_FULL = 7812
TAIL = 64


def _make_transpose():
    mesh = plsc.VectorSubcoreMesh(core_axis_name="c", subcore_axis_name="s")
    out_type = jax.ShapeDtypeStruct((16000000,), jnp.float32)

    @functools.partial(
        pl.kernel,
        mesh=mesh,
        out_type=out_type,
        compiler_params=pltpu.CompilerParams(
            use_tc_tiling_on_sc=True, needs_layout_passes=False
        ),
        scratch_types=[
            pltpu.VMEM((D, 128), jnp.float32),
            pltpu.VMEM((D, 128), jnp.float32),
            pltpu.VMEM((128 * D,), jnp.float32),
            pltpu.VMEM((128 * D,), jnp.float32),
            pltpu.SemaphoreType.DMA,
            pltpu.SemaphoreType.DMA,
        ],
    )
    def tr_kernel(tT_hbm, tail_hbm, out_hbm, buf0, buf1, tp0, tp1, sem_i, sem_o):
        bufs = (buf0, buf1)
        tps = (tp0, tp1)
        wid = lax.axis_index("s") * 2 + lax.axis_index("c")
        nblk = jnp.where(wid < 4, 245, 244)
        start = wid * 244 + jnp.minimum(wid, 4)
        iota16 = lax.iota(jnp.int32, 16) * D

        def fire_in(b, slot):
            pltpu.async_copy(
                tT_hbm.at[:, pl.ds(b * 128, 128)], bufs[slot], sem_i
            )

        def wait_in(slot):
            pltpu.make_async_copy(
                tT_hbm.at[:, pl.ds(0, 128)], bufs[slot], sem_i
            ).wait()

        def transpose(slot):
            for jj in range(8):
                for f in range(D):
                    vec = bufs[slot][f, pl.ds(jj * 16, 16)]
                    pos = iota16 + (jj * 256 + f)
                    plsc.store_scatter(tps[slot], [pos], vec)

        def fire_out(b, slot):
            pltpu.async_copy(
                tps[slot], out_hbm.at[pl.ds(b * 2048, 2048)], sem_o
            )

        def wait_out(slot):
            pltpu.make_async_copy(
                tps[slot], out_hbm.at[pl.ds(0, 2048)], sem_o
            ).wait()

        fire_in(start, 0)

        def body(i, _):
            for sl in range(NBUF):
                k = NBUF * i + sl
                b = start + k

                @pl.when(k < nblk)
                def _():
                    @pl.when(k + 1 < nblk)
                    def _():
                        fire_in(b + 1, 1 - sl)

                    wait_in(sl)

                    @pl.when(k >= 2)
                    def _():
                        wait_out(sl)

                    transpose(sl)
                    fire_out(b, sl)

            return ()

        lax.fori_loop(0, 123, body, ())
        wait_out(0)
        wait_out(0)

        @pl.when(wid == 31)
        def _():
            pltpu.sync_copy(tail_hbm, tp0.at[pl.ds(0, TAIL * D)])
            pltpu.sync_copy(
                tp0.at[pl.ds(0, TAIL * D)],
                out_hbm.at[pl.ds(NB_FULL * 2048, TAIL * D)],
            )

    return tr_kernel


def _make_gather():
    mesh = plsc.VectorSubcoreMesh(core_axis_name="c", subcore_axis_name="s")
    out_type = jax.ShapeDtypeStruct((S, 2, NW * G, 8, RG), jnp.float32)

    @functools.partial(
        pl.kernel,
        mesh=mesh,
        out_type=out_type,
        compiler_params=pltpu.CompilerParams(
            use_tc_tiling_on_sc=False, needs_layout_passes=False
        ),
        scratch_types=[
            pltpu.VMEM((ROWS_W, S), jnp.int32),
            pltpu.VMEM((NBUF, RG * S, D), jnp.float32),
            pltpu.VMEM((D, RG), jnp.float32),
            pltpu.VMEM((D, RG), jnp.float32),
            pltpu.SemaphoreType.DMA,
            pltpu.SemaphoreType.DMA,
            pltpu.SemaphoreType.DMA,
            pltpu.SemaphoreType.DMA,
        ],
    )
    def gather_kernel(
        idx_hbm, table_hbm, out_hbm, idx_v, rows_v, blk0, blk1, sem0, sem1, semw0, semw1
    ):
        blks = (blk0, blk1)
        semws = (semw0, semw1)
        wid = lax.axis_index("s") * 2 + lax.axis_index("c")
        base = wid * ROWS_W
        pltpu.sync_copy(idx_hbm.at[pl.ds(base, ROWS_W)], idx_v)
        sems = (sem0, sem1)

        iota = lax.iota(jnp.int32, 16)

        def fire(g, slot, sem):
            def frow(r, _):
                pltpu.async_copy(
                    table_hbm.at[idx_v.at[g * RG + r]],
                    rows_v.at[slot, pl.ds(r * S, S)],
                    sem,
                )
                return ()

            lax.fori_loop(0, RG, frow, ())

        def drain(slot, sem):
            pltpu.make_async_copy(
                table_hbm.at[pl.ds(0, RG * S)], rows_v.at[slot], sem
            ).wait()

        def wait_emit(par):
            for half in range(2):
                pltpu.make_async_copy(
                    blks[par].at[pl.ds(half * 8, 8)],
                    out_hbm.at[0, 0, 0],
                    semws[par],
                ).wait()

        def emit(g, slot):
            bt = wid * G + g

            def fs(i2, _):
                for par in range(2):
                    s = 2 * i2 + par
                    blk = blks[par]

                    @pl.when(g * S + s >= 2)
                    def _():
                        wait_emit(par)

                    def fb(bo, _):
                        for j in range(8):
                            b = bo * 8 + j
                            vec = rows_v[slot, b * S + s, :]
                            bvec = jnp.full((16,), b, jnp.int32)
                            plsc.store_scatter(blk, [iota, bvec], vec)
                        return ()

                    lax.fori_loop(0, RG // 8, fb, ())
                    pltpu.async_copy(
                        blk.at[pl.ds(0, 8)], out_hbm.at[s, 0, bt], semws[par]
                    )
                    pltpu.async_copy(
                        blk.at[pl.ds(8, 8)], out_hbm.at[s, 1, bt], semws[par]
                    )
                return ()

            lax.fori_loop(0, S // 2, fs, ())

        fire(0, 0, sem0)

        def body(i, _):
            for b in range(NBUF):
                g = NBUF * i + b
                nb = 1 - b

                @pl.when(g + 1 < G)
                def _():
                    fire(g + 1, nb, sems[nb])

                drain(b, sems[b])
                emit(g, b)
            return ()

        lax.fori_loop(0, G // NBUF, body, ())
        wait_emit(0)
        wait_emit(1)

    return gather_kernel


_transpose = _make_transpose()
_gather = _make_gather()


def kernel(index, table):
    tail = table[NB_FULL * 128 :].reshape(TAIL * D)
    t_lin = _transpose(table.T, tail)
    out5d = _gather(index.astype(jnp.int32), t_lin.reshape(1000000, 16))
    o = jnp.transpose(out5d, (2, 4, 0, 1, 3))
    return o.reshape(16384, S, D)


# speedup vs baseline: 3.0081x; 1.1509x over previous
import functools

import jax
import jax.numpy as jnp
from jax import lax
from jax.experimental import pallas as pl
from jax.experimental.pallas import tpu as pltpu
from jax.experimental.pallas import tpu_sc as plsc

D = 16
S = 26
NW = 32
ROWS_W = 512
RG = 128
G = ROWS_W // RG
NBUF = 2
NB_FULL = 7812
TAIL = 64


def _make_transpose():
    mesh = plsc.VectorSubcoreMesh(core_axis_name="c", subcore_axis_name="s")
    out_type = jax.ShapeDtypeStruct((16000000,), jnp.float32)

    @functools.partial(
        pl.kernel,
        mesh=mesh,
        out_type=out_type,
        compiler_params=pltpu.CompilerParams(
            use_tc_tiling_on_sc=True, needs_layout_passes=False
        ),
        scratch_types=[
            pltpu.VMEM((D, 128), jnp.float32),
            pltpu.VMEM((D, 128), jnp.float32),
            pltpu.VMEM((D, 128), jnp.float32),
            pltpu.VMEM((D, 128), jnp.float32),
            pltpu.VMEM((128 * D,), jnp.float32),
            pltpu.VMEM((128 * D,), jnp.float32),
            pltpu.VMEM((128 * D,), jnp.float32),
            pltpu.VMEM((128 * D,), jnp.float32),
            pltpu.SemaphoreType.DMA,
            pltpu.SemaphoreType.DMA,
        ],
    )
    def tr_kernel(
        tT_hbm, tail_hbm, out_hbm,
        buf0, buf1, buf2, buf3, tp0, tp1, tp2, tp3, sem_i, sem_o,
    ):
        bufs = (buf0, buf1, buf2, buf3)
        tps = (tp0, tp1, tp2, tp3)
        wid = lax.axis_index("s") * 2 + lax.axis_index("c")
        nblk = jnp.where(wid < 4, 245, 244)
        start = wid * 244 + jnp.minimum(wid, 4)
        iota16 = lax.iota(jnp.int32, 16) * D

        def fire_in(b, slot):
            pltpu.async_copy(
                tT_hbm.at[:, pl.ds(b * 128, 128)], bufs[slot], sem_i
            )

        def wait_in(slot):
            pltpu.make_async_copy(
                tT_hbm.at[:, pl.ds(0, 128)], bufs[slot], sem_i
            ).wait()

        def transpose(slot):
            for jj in range(8):
                for f in range(D):
                    vec = bufs[slot][f, pl.ds(jj * 16, 16)]
                    pos = iota16 + (jj * 256 + f)
                    plsc.store_scatter(tps[slot], [pos], vec)

        def fire_out(b, slot):
            pltpu.async_copy(
                tps[slot], out_hbm.at[pl.ds(b * 2048, 2048)], sem_o
            )

        def wait_out(slot):
            pltpu.make_async_copy(
                tps[slot], out_hbm.at[pl.ds(0, 2048)], sem_o
            ).wait()

        for p in range(3):
            fire_in(start + p, p)

        def body(i, _):
            for sl in range(4):
                k = 4 * i + sl
                b = start + k

                @pl.when(k < nblk)
                def _():
                    @pl.when(k + 3 < nblk)
                    def _():
                        fire_in(b + 3, (sl + 3) % 4)

                    wait_in(sl)

                    @pl.when(k >= 4)
                    def _():
                        wait_out(sl)

                    transpose(sl)
                    fire_out(b, sl)

            return ()

        lax.fori_loop(0, 62, body, ())
        for _p in range(4):
            wait_out(0)

        @pl.when(wid == 31)
        def _():
            pltpu.sync_copy(tail_hbm, tp0.at[pl.ds(0, TAIL * D)])
            pltpu.sync_copy(
                tp0.at[pl.ds(0, TAIL * D)],
                out_hbm.at[pl.ds(NB_FULL * 2048, TAIL * D)],
            )

    return tr_kernel


def _make_gather():
    mesh = plsc.VectorSubcoreMesh(core_axis_name="c", subcore_axis_name="s")
    out_type = jax.ShapeDtypeStruct((S, 2, NW * G, 8, RG), jnp.float32)

    @functools.partial(
        pl.kernel,
        mesh=mesh,
        out_type=out_type,
        compiler_params=pltpu.CompilerParams(
            use_tc_tiling_on_sc=False, needs_layout_passes=False
        ),
        scratch_types=[
            pltpu.VMEM((ROWS_W, S), jnp.int32),
            pltpu.VMEM((NBUF, RG * S, D), jnp.float32),
            pltpu.VMEM((D, RG), jnp.float32),
            pltpu.VMEM((D, RG), jnp.float32),
            pltpu.SemaphoreType.DMA,
            pltpu.SemaphoreType.DMA,
            pltpu.SemaphoreType.DMA,
            pltpu.SemaphoreType.DMA,
        ],
    )
    def gather_kernel(
        idx_hbm, table_hbm, out_hbm, idx_v, rows_v, blk0, blk1, sem0, sem1, semw0, semw1
    ):
        blks = (blk0, blk1)
        semws = (semw0, semw1)
        wid = lax.axis_index("s") * 2 + lax.axis_index("c")
        base = wid * ROWS_W
        pltpu.sync_copy(idx_hbm.at[pl.ds(base, ROWS_W)], idx_v)
        sems = (sem0, sem1)

        iota = lax.iota(jnp.int32, 16)

        def fire(g, slot, sem):
            def frow(r, _):
                pltpu.async_copy(
                    table_hbm.at[idx_v.at[g * RG + r]],
                    rows_v.at[slot, pl.ds(r * S, S)],
                    sem,
                )
                return ()

            lax.fori_loop(0, RG, frow, ())

        def drain(slot, sem):
            pltpu.make_async_copy(
                table_hbm.at[pl.ds(0, RG * S)], rows_v.at[slot], sem
            ).wait()

        def wait_emit(par):
            for half in range(2):
                pltpu.make_async_copy(
                    blks[par].at[pl.ds(half * 8, 8)],
                    out_hbm.at[0, 0, 0],
                    semws[par],
                ).wait()

        def emit(g, slot):
            bt = wid * G + g

            def fs(i2, _):
                for par in range(2):
                    s = 2 * i2 + par
                    blk = blks[par]

                    @pl.when(g * S + s >= 2)
                    def _():
                        wait_emit(par)

                    def fb(bo, _):
                        for j in range(8):
                            b = bo * 8 + j
                            vec = rows_v[slot, b * S + s, :]
                            bvec = jnp.full((16,), b, jnp.int32)
                            plsc.store_scatter(blk, [iota, bvec], vec)
                        return ()

                    lax.fori_loop(0, RG // 8, fb, ())
                    pltpu.async_copy(
                        blk.at[pl.ds(0, 8)], out_hbm.at[s, 0, bt], semws[par]
                    )
                    pltpu.async_copy(
                        blk.at[pl.ds(8, 8)], out_hbm.at[s, 1, bt], semws[par]
                    )
                return ()

            lax.fori_loop(0, S // 2, fs, ())

        fire(0, 0, sem0)

        def body(i, _):
            for b in range(NBUF):
                g = NBUF * i + b
                nb = 1 - b

                @pl.when(g + 1 < G)
                def _():
                    fire(g + 1, nb, sems[nb])

                drain(b, sems[b])
                emit(g, b)
            return ()

        lax.fori_loop(0, G // NBUF, body, ())
        wait_emit(0)
        wait_emit(1)

    return gather_kernel


_transpose = _make_transpose()
_gather = _make_gather()


def kernel(index, table):
    tail = table[NB_FULL * 128 :].reshape(TAIL * D)
    t_lin = _transpose(table.T, tail)
    out5d = _gather(index.astype(jnp.int32), t_lin.reshape(1000000, 16))
    o = jnp.transpose(out5d, (2, 4, 0, 1, 3))
    return o.reshape(16384, S, D)
